# baseline (speedup 1.0000x reference)
import functools
import math

import jax
import jax.numpy as jnp
from jax import lax
from jax.experimental import pallas as pl
from jax.experimental.pallas import tpu as pltpu
from jax.experimental.pallas import tpu_sc as plsc


def _sc_geometry():
    try:
        info = plsc.get_sparse_core_info()
        return info.num_cores, info.num_subcores
    except Exception:
        return 2, 16


@functools.lru_cache(maxsize=None)
def _build(BATCH, SEQ, V, D):
    NC, NS = _sc_geometry()
    NW = NC * NS
    assert BATCH % NW == 0
    rows_per_w = BATCH // NW
    NBUF = 4
    scale = math.sqrt(D)
    assert D % 16 == 0
    d_vecs = D // 16
    assert 128 < SEQ <= 256
    REM = SEQ - 128
    assert rows_per_w % NBUF == 0
    n_steps = rows_per_w // NBUF

    mesh = plsc.VectorSubcoreMesh(core_axis_name="c", subcore_axis_name="s")

    @functools.partial(
        pl.kernel,
        out_type=jax.ShapeDtypeStruct((BATCH * SEQ, 2 * D), jnp.float32),
        mesh=mesh,
        scratch_types=[
            pltpu.VMEM((rows_per_w, SEQ), jnp.int32),
            [pltpu.VMEM((SEQ,), jnp.int32) for _ in range(NBUF)],
            [pltpu.VMEM((SEQ, D), jnp.float32) for _ in range(NBUF)],
            [pltpu.VMEM((SEQ, D), jnp.float32) for _ in range(NBUF)],
            [pltpu.SemaphoreType.DMA for _ in range(NBUF)],
            [pltpu.SemaphoreType.DMA for _ in range(NBUF)],
        ],
        compiler_params=pltpu.CompilerParams(use_tc_tiling_on_sc=False),
    )
    def emb_kernel(tokens_hbm, table_hbm, out_hbm, idx_v, idx2, rows_g,
                   rows_s, sem_g, sem_s):
        wid = lax.axis_index("s") * NC + lax.axis_index("c")
        row0 = wid * rows_per_w

        pltpu.sync_copy(tokens_hbm.at[pl.ds(row0, rows_per_w)], idx_v)

        def start_gather(r, b):
            starts = [16 * k for k in range(SEQ // 16)] + [SEQ - 16]
            for c in starts:
                sl = pl.ds(c, 16)
                idx2[b][sl] = idx_v[r, sl] * 2
            pltpu.async_copy(table_hbm.at[idx2[b]], rows_g[b], sem_g[b])

        def wait_gather(b):
            pltpu.make_async_copy(
                table_hbm.at[idx2[b]], rows_g[b], sem_g[b]).wait()

        def start_scatter(r, b):
            pltpu.async_copy(
                rows_s[b],
                out_hbm.at[pl.ds((row0 + r) * SEQ, SEQ), pl.ds(0, D)],
                sem_s[b])

        def wait_scatter(b):
            pltpu.make_async_copy(
                rows_s[b], out_hbm.at[pl.ds(0, SEQ), pl.ds(0, D)],
                sem_s[b]).wait()

        for b in range(NBUF):
            start_gather(b, b)

        def step_body(step, _):
            for b in range(NBUF):
                r = step * NBUF + b
                wait_gather(b)

                @pl.when(step > 0)
                def _wait_prev_scatter(b=b):
                    wait_scatter(b)

                src, dst = rows_g[b], rows_s[b]

                @pl.loop(0, SEQ, unroll=8)
                def _scale(i):
                    for j in range(d_vecs):
                        sl = pl.ds(j * 16, 16)
                        dst[i, sl] = src[i, sl] * scale

                start_scatter(r, b)

                r2 = r + NBUF

                @pl.when(r2 < rows_per_w)
                def _next_gather(r2=r2, b=b):
                    start_gather(r2, b)
            return ()

        lax.fori_loop(0, n_steps, step_body, ())

        for b in range(NBUF):
            wait_scatter(b)

    return emb_kernel


def kernel(tokens, table):
    batch, seq = tokens.shape
    V, D = table.shape
    table2 = jnp.pad(table, ((0, 0), (0, D))).reshape(2 * V, D)
    out = _build(batch, seq, V, D)(tokens.astype(jnp.int32), table2)
    return out.reshape(batch, seq, 2 * D)[:, :, :D]


# speedup vs baseline: 1.9159x; 1.9159x over previous
import functools
import math

import jax
import jax.numpy as jnp
from jax import lax
from jax.experimental import pallas as pl
from jax.experimental.pallas import tpu as pltpu
from jax.experimental.pallas import tpu_sc as plsc


def _sc_geometry():
    try:
        info = plsc.get_sparse_core_info()
        return info.num_cores, info.num_subcores
    except Exception:
        return 2, 16


def _make_repack(V, D, scale):
    BLK = 6400
    grid = -(-V // BLK)

    def _repack_body(x_ref, o_ref):
        o_ref[:, :D] = x_ref[...].T * scale

    return pl.pallas_call(
        _repack_body,
        grid=(grid,),
        in_specs=[pl.BlockSpec((D, BLK), lambda i: (0, i))],
        out_specs=pl.BlockSpec((BLK, 2 * D), lambda i: (i, 0)),
        out_shape=jax.ShapeDtypeStruct((V, 2 * D), jnp.float32),
    )


@functools.lru_cache(maxsize=None)
def _build(BATCH, SEQ, V, D):
    NC, NS = _sc_geometry()
    NW = NC * NS
    assert BATCH % NW == 0
    rows_per_w = BATCH // NW
    NBUF = 8
    LAG = 4
    assert rows_per_w % NBUF == 0
    assert D % 16 == 0
    assert SEQ % 16 == 0 or True

    mesh = plsc.VectorSubcoreMesh(core_axis_name="c", subcore_axis_name="s")

    @functools.partial(
        pl.kernel,
        out_type=jax.ShapeDtypeStruct((BATCH * SEQ, 2 * D), jnp.float32),
        mesh=mesh,
        scratch_types=[
            pltpu.VMEM((rows_per_w, SEQ), jnp.int32),
            [pltpu.VMEM((SEQ,), jnp.int32) for _ in range(NBUF)],
            [pltpu.VMEM((SEQ, D), jnp.float32) for _ in range(NBUF)],
            [pltpu.SemaphoreType.DMA for _ in range(NBUF)],
            [pltpu.SemaphoreType.DMA for _ in range(NBUF)],
        ],
        compiler_params=pltpu.CompilerParams(use_tc_tiling_on_sc=False),
    )
    def emb_kernel(tokens_hbm, table_hbm, out_hbm, idx_v, idx2, rows_g,
                   sem_g, sem_s):
        wid = lax.axis_index("s") * NC + lax.axis_index("c")
        row0 = wid * rows_per_w

        pltpu.sync_copy(tokens_hbm.at[pl.ds(row0, rows_per_w)], idx_v)

        def start_gather(r, b):
            starts = [16 * k for k in range(SEQ // 16)]
            if SEQ % 16:
                starts.append(SEQ - 16)
            for c in starts:
                sl = pl.ds(c, 16)
                idx2[b][sl] = idx_v[r, sl] * 2
            pltpu.async_copy(table_hbm.at[idx2[b]], rows_g[b], sem_g[b])

        def wait_gather(b):
            pltpu.make_async_copy(
                table_hbm.at[idx2[b]], rows_g[b], sem_g[b]).wait()

        def start_scatter(r, b):
            pltpu.async_copy(
                rows_g[b],
                out_hbm.at[pl.ds((row0 + r) * SEQ, SEQ), pl.ds(0, D)],
                sem_s[b])

        def wait_scatter(b):
            pltpu.make_async_copy(
                rows_g[b], out_hbm.at[pl.ds(0, SEQ), pl.ds(0, D)],
                sem_s[b]).wait()

        for r in range(LAG):
            start_gather(r, r % NBUF)

        def step_body(step, _):
            for j in range(NBUF):
                r = step * NBUF + j
                b = j
                wait_gather(b)
                start_scatter(r, b)
                b2 = (j + LAG) % NBUF

                @pl.when(r + LAG < rows_per_w)
                def _next(r=r, b2=b2):
                    @pl.when(r - LAG >= 0)
                    def _drain(b2=b2):
                        wait_scatter(b2)

                    start_gather(r + LAG, b2)
            return ()

        lax.fori_loop(0, rows_per_w // NBUF, step_body, ())

        for r in range(rows_per_w - NBUF, rows_per_w):
            wait_scatter(r % NBUF)

    return emb_kernel


def kernel(tokens, table):
    batch, seq = tokens.shape
    V, D = table.shape
    scale = math.sqrt(D)
    t8 = _make_repack(V, D, scale)(table.T).reshape(2 * V, D)
    out = _build(batch, seq, V, D)(tokens.astype(jnp.int32), t8)
    return out.reshape(batch, seq, 2 * D)[:, :, :D]


# speedup vs baseline: 1.9952x; 1.0414x over previous
import functools
import math

import jax
import jax.numpy as jnp
from jax import lax
from jax.experimental import pallas as pl
from jax.experimental.pallas import tpu as pltpu
from jax.experimental.pallas import tpu_sc as plsc


def _sc_geometry():
    try:
        info = plsc.get_sparse_core_info()
        return info.num_cores, info.num_subcores
    except Exception:
        return 2, 16


def _make_repack(V, D, scale):
    BLK = 8192
    HALF = BLK // 2
    grid = -(-V // BLK)

    def _repack_body(x_ref, o_ref):
        x = x_ref[...]
        o_ref[:, :D] = x[:, :HALF].T * scale
        o_ref[:, D:] = x[:, HALF:].T * scale

    return pl.pallas_call(
        _repack_body,
        grid=(grid,),
        in_specs=[pl.BlockSpec((D, BLK), lambda i: (0, i))],
        out_specs=pl.BlockSpec((HALF, 2 * D), lambda i: (i, 0)),
        out_shape=jax.ShapeDtypeStruct((grid * HALF, 2 * D), jnp.float32),
    )


@functools.lru_cache(maxsize=None)
def _build(BATCH, SEQ, V, D):
    NC, NS = _sc_geometry()
    NW = NC * NS
    assert BATCH % NW == 0
    rows_per_w = BATCH // NW
    NBUF = 8
    LAG = 4
    BLK = 8192
    assert rows_per_w % NBUF == 0
    assert D % 16 == 0

    mesh = plsc.VectorSubcoreMesh(core_axis_name="c", subcore_axis_name="s")

    @functools.partial(
        pl.kernel,
        out_type=jax.ShapeDtypeStruct((BATCH * SEQ, 2 * D), jnp.float32),
        mesh=mesh,
        scratch_types=[
            pltpu.VMEM((rows_per_w, SEQ), jnp.int32),
            [pltpu.VMEM((SEQ,), jnp.int32) for _ in range(NBUF)],
            [pltpu.VMEM((SEQ, D), jnp.float32) for _ in range(NBUF)],
            [pltpu.SemaphoreType.DMA for _ in range(NBUF)],
            [pltpu.SemaphoreType.DMA for _ in range(NBUF)],
        ],
        compiler_params=pltpu.CompilerParams(use_tc_tiling_on_sc=False),
    )
    def emb_kernel(tokens_hbm, table_hbm, out_hbm, idx_v, idx2, rows_g,
                   sem_g, sem_s):
        wid = lax.axis_index("s") * NC + lax.axis_index("c")
        row0 = wid * rows_per_w

        pltpu.sync_copy(tokens_hbm.at[pl.ds(row0, rows_per_w)], idx_v)

        def start_gather(r, b):
            starts = [16 * k for k in range(SEQ // 16)]
            if SEQ % 16:
                starts.append(SEQ - 16)
            for c in starts:
                sl = pl.ds(c, 16)
                x = idx_v[r, sl]
                off = x & (BLK - 1)
                base = x + off
                idx2[b][sl] = jnp.where(off >= BLK // 2, base - (BLK - 1),
                                        base)
            pltpu.async_copy(table_hbm.at[idx2[b]], rows_g[b], sem_g[b])

        def wait_gather(b):
            pltpu.make_async_copy(
                table_hbm.at[idx2[b]], rows_g[b], sem_g[b]).wait()

        def start_scatter(r, b):
            pltpu.async_copy(
                rows_g[b],
                out_hbm.at[pl.ds((row0 + r) * SEQ, SEQ), pl.ds(0, D)],
                sem_s[b])

        def wait_scatter(b):
            pltpu.make_async_copy(
                rows_g[b], out_hbm.at[pl.ds(0, SEQ), pl.ds(0, D)],
                sem_s[b]).wait()

        for r in range(LAG):
            start_gather(r, r % NBUF)

        def step_body(step, _):
            for j in range(NBUF):
                r = step * NBUF + j
                b = j
                wait_gather(b)
                start_scatter(r, b)
                b2 = (j + LAG) % NBUF

                @pl.when(r + LAG < rows_per_w)
                def _next(r=r, b2=b2):
                    @pl.when(r - LAG >= 0)
                    def _drain(b2=b2):
                        wait_scatter(b2)

                    start_gather(r + LAG, b2)
            return ()

        lax.fori_loop(0, rows_per_w // NBUF, step_body, ())

        for r in range(rows_per_w - NBUF, rows_per_w):
            wait_scatter(r % NBUF)

    return emb_kernel


def kernel(tokens, table):
    batch, seq = tokens.shape
    V, D = table.shape
    scale = math.sqrt(D)
    t8 = _make_repack(V, D, scale)(table.T)
    t8 = t8.reshape(2 * t8.shape[0], D)
    out = _build(batch, seq, V, D)(tokens.astype(jnp.int32), t8)
    return out.reshape(batch, seq, 2 * D)[:, :, :D]


# speedup vs baseline: 2.1689x; 1.0871x over previous
import functools
import math

import jax
import jax.numpy as jnp
from jax import lax
from jax.experimental import pallas as pl
from jax.experimental.pallas import tpu as pltpu
from jax.experimental.pallas import tpu_sc as plsc


def _sc_geometry():
    try:
        info = plsc.get_sparse_core_info()
        return info.num_cores, info.num_subcores
    except Exception:
        return 2, 16


def _make_repack(V, D, scale):
    BLK = 8192
    HALF = BLK // 2
    grid = -(-V // BLK)

    def _repack_body(x_ref, o_ref):
        x = x_ref[...]
        o_ref[:, :D] = x[:, :HALF].T * scale
        o_ref[:, D:] = x[:, HALF:].T * scale

    return pl.pallas_call(
        _repack_body,
        grid=(grid,),
        in_specs=[pl.BlockSpec((D, BLK), lambda i: (0, i))],
        out_specs=pl.BlockSpec((HALF, 2 * D), lambda i: (i, 0)),
        out_shape=jax.ShapeDtypeStruct((grid * HALF, 2 * D), jnp.float32),
        compiler_params=pltpu.CompilerParams(
            vmem_limit_bytes=120 * 1024 * 1024),
    )


@functools.lru_cache(maxsize=None)
def _build(BATCH, SEQ, V, D):
    NC, NS = _sc_geometry()
    NW = NC * NS
    assert BATCH % NW == 0
    rows_per_w = BATCH // NW
    NBUF = 8
    LAG = 4
    BLK = 8192
    assert rows_per_w % NBUF == 0
    assert D % 16 == 0

    mesh = plsc.VectorSubcoreMesh(core_axis_name="c", subcore_axis_name="s")

    @functools.partial(
        pl.kernel,
        out_type=jax.ShapeDtypeStruct((BATCH * SEQ, 2 * D), jnp.float32),
        mesh=mesh,
        scratch_types=[
            pltpu.VMEM((rows_per_w, SEQ), jnp.int32),
            [pltpu.VMEM((SEQ,), jnp.int32) for _ in range(NBUF)],
            [pltpu.VMEM((SEQ, D), jnp.float32) for _ in range(NBUF)],
            [pltpu.SemaphoreType.DMA for _ in range(NBUF)],
            [pltpu.SemaphoreType.DMA for _ in range(NBUF)],
        ],
        compiler_params=pltpu.CompilerParams(use_tc_tiling_on_sc=False),
    )
    def emb_kernel(tokens_hbm, table_hbm, out_hbm, idx_v, idx2, rows_g,
                   sem_g, sem_s):
        wid = lax.axis_index("s") * NC + lax.axis_index("c")
        row0 = wid * rows_per_w

        pltpu.sync_copy(tokens_hbm.at[pl.ds(row0, rows_per_w)], idx_v)

        def start_gather(r, b):
            starts = [16 * k for k in range(SEQ // 16)]
            if SEQ % 16:
                starts.append(SEQ - 16)
            for c in starts:
                sl = pl.ds(c, 16)
                x = idx_v[r, sl]
                off = x & (BLK - 1)
                base = x + off
                idx2[b][sl] = jnp.where(off >= BLK // 2, base - (BLK - 1),
                                        base)
            pltpu.async_copy(table_hbm.at[idx2[b]], rows_g[b], sem_g[b])

        def wait_gather(b):
            pltpu.make_async_copy(
                table_hbm.at[idx2[b]], rows_g[b], sem_g[b]).wait()

        def start_scatter(r, b):
            pltpu.async_copy(
                rows_g[b],
                out_hbm.at[pl.ds((row0 + r) * SEQ, SEQ), pl.ds(0, D)],
                sem_s[b])

        def wait_scatter(b):
            pltpu.make_async_copy(
                rows_g[b], out_hbm.at[pl.ds(0, SEQ), pl.ds(0, D)],
                sem_s[b]).wait()

        for r in range(LAG):
            start_gather(r, r % NBUF)

        def step_body(step, _):
            for j in range(NBUF):
                r = step * NBUF + j
                b = j
                wait_gather(b)
                start_scatter(r, b)
                b2 = (j + LAG) % NBUF

                @pl.when(r + LAG < rows_per_w)
                def _next(r=r, b2=b2):
                    @pl.when(r - LAG >= 0)
                    def _drain(b2=b2):
                        wait_scatter(b2)

                    start_gather(r + LAG, b2)
            return ()

        lax.fori_loop(0, rows_per_w // NBUF, step_body, ())

        for r in range(rows_per_w - NBUF, rows_per_w):
            wait_scatter(r % NBUF)

    return emb_kernel


def kernel(tokens, table):
    batch, seq = tokens.shape
    V, D = table.shape
    scale = math.sqrt(D)
    t8 = _make_repack(V, D, scale)(table.T)
    t8 = t8.reshape(2 * t8.shape[0], D)
    out = _build(batch, seq, V, D)(tokens.astype(jnp.int32), t8)
    return out.reshape(batch, seq, 2 * D)[:, :, :D]
